# Initial kernel scaffold; baseline (speedup 1.0000x reference)
import jax
import jax.numpy as jnp
from jax.experimental import pallas as pl


def kernel(h, pos, segment_ids):
    raise NotImplementedError("write your pallas kernel here")


# speedup vs baseline: 6.1719x; 6.1719x over previous
import functools

import jax
import jax.numpy as jnp
from jax import lax
from jax.experimental import pallas as pl
from jax.experimental.pallas import tpu as pltpu
from jax.experimental.pallas import tpu_sc as plsc

N = 100000
D = 128
B = 256
NPOS = 3
ROWS = 80
NB = N // ROWS
NC, NS = 2, 16
NW = NC * NS
ACC_ROWS = B * NPOS
STRIPE = ACC_ROWS // NS


def _sc_body(h_hbm, pos_hbm, seg_hbm, out_hbm, cnt_out_hbm,
             seg_v, pos_v, rows_v, ones_v, zbuf_v,
             acc_sh, cnt_sh):
    cid = lax.axis_index("c")
    sid = lax.axis_index("s")
    wid = sid * NC + cid

    zeros16 = jnp.zeros((16,), jnp.float32)
    ones16 = jnp.ones((16,), jnp.float32)

    def _zrow(r, carry):
        for q in range(D // 16):
            zbuf_v[r, pl.ds(16 * q, 16)] = zeros16
        return carry
    lax.fori_loop(0, STRIPE, _zrow, 0)

    for r in range(16):
        for q in range(D // 16):
            ones_v[r, pl.ds(16 * q, 16)] = ones16

    pltpu.sync_copy(zbuf_v, acc_sh.at[pl.ds(sid * STRIPE, STRIPE)])
    pltpu.sync_copy(zbuf_v.at[pl.ds(0, 16)], cnt_sh.at[pl.ds(sid * 16, 16)])

    plsc.subcore_barrier()

    nb_w = (NB - 1 - wid) // NW + 1

    def _batch(j, carry):
        base = (wid + j * NW) * ROWS
        pltpu.sync_copy(seg_hbm.at[pl.ds(base, ROWS)], seg_v)
        pltpu.sync_copy(pos_hbm.at[pl.ds(base, ROWS)], pos_v)
        pltpu.sync_copy(h_hbm.at[pl.ds(base, ROWS)], rows_v)
        for i in range(ROWS // 16):
            s = seg_v[pl.ds(16 * i, 16)]
            p = pos_v[pl.ds(16 * i, 16)]
            t = s * NPOS + p
            pltpu.sync_copy(rows_v.at[pl.ds(16 * i, 16)],
                            acc_sh.at[t], add=True)
            pltpu.sync_copy(ones_v, cnt_sh.at[s], add=True)
        return carry

    lax.fori_loop(0, nb_w, _batch, 0)

    plsc.subcore_barrier()

    pltpu.sync_copy(acc_sh.at[pl.ds(sid * STRIPE, STRIPE)],
                    out_hbm.at[cid, pl.ds(sid * STRIPE, STRIPE)])
    pltpu.sync_copy(cnt_sh.at[pl.ds(sid * 16, 16)],
                    cnt_out_hbm.at[cid, pl.ds(sid * 16, 16)])


_sc_call = functools.partial(
    pl.kernel,
    out_type=(
        jax.ShapeDtypeStruct((NC, ACC_ROWS, D), jnp.float32),
        jax.ShapeDtypeStruct((NC, B, D), jnp.float32),
    ),
    mesh=plsc.VectorSubcoreMesh(
        core_axis_name="c", subcore_axis_name="s",
        num_cores=NC, num_subcores=NS),
    scratch_types=[
        pltpu.VMEM((ROWS,), jnp.int32),
        pltpu.VMEM((ROWS,), jnp.int32),
        pltpu.VMEM((ROWS, D), jnp.float32),
        pltpu.VMEM((16, D), jnp.float32),
        pltpu.VMEM((STRIPE, D), jnp.float32),
        pltpu.VMEM_SHARED((ACC_ROWS, D), jnp.float32),
        pltpu.VMEM_SHARED((B, D), jnp.float32),
    ],
)(_sc_body)


def kernel(h, pos, segment_ids):
    partial, cnt = _sc_call(h, pos, segment_ids)
    acc = partial[0] + partial[1]
    counts = cnt[0, :, 0] + cnt[1, :, 0]
    norm = jnp.maximum(counts, 1.0)
    out = acc.reshape(B, NPOS, D) / norm[:, None, None]
    return out.reshape(B, NPOS * D)


# speedup vs baseline: 6.9782x; 1.1306x over previous
import functools

import jax
import jax.numpy as jnp
from jax import lax
from jax.experimental import pallas as pl
from jax.experimental.pallas import tpu as pltpu
from jax.experimental.pallas import tpu_sc as plsc

N = 100000
D = 128
B = 256
NPOS = 3
ROWS = 80
NB = N // ROWS
NC, NS = 2, 16
NW = NC * NS
ACC_ROWS = B * NPOS
STRIPE = ACC_ROWS // NS


def _sc_body(h_hbm, pos_hbm, seg_hbm, out_hbm, cnt_out_hbm,
             seg_v, pos_v, idx_v, rows_v, ones_v, zbuf_v,
             acc_sh, cnt_sh):
    cid = lax.axis_index("c")
    sid = lax.axis_index("s")
    wid = sid * NC + cid

    zeros16 = jnp.zeros((16,), jnp.float32)
    ones16 = jnp.ones((16,), jnp.float32)

    def _zrow(r, carry):
        for q in range(D // 16):
            zbuf_v[r, pl.ds(16 * q, 16)] = zeros16
        return carry
    lax.fori_loop(0, STRIPE, _zrow, 0)

    def _orow(r, carry):
        for q in range(D // 16):
            ones_v[r, pl.ds(16 * q, 16)] = ones16
        return carry
    lax.fori_loop(0, ROWS, _orow, 0)

    pltpu.sync_copy(zbuf_v, acc_sh.at[pl.ds(sid * STRIPE, STRIPE)])
    pltpu.sync_copy(zbuf_v.at[pl.ds(0, 16)], cnt_sh.at[pl.ds(sid * 16, 16)])

    plsc.subcore_barrier()

    nb_w = (NB - 1 - wid) // NW + 1

    def _batch(j, carry):
        base = (wid + j * NW) * ROWS
        pltpu.sync_copy(seg_hbm.at[pl.ds(base, ROWS)], seg_v)
        pltpu.sync_copy(pos_hbm.at[pl.ds(base, ROWS)], pos_v)
        pltpu.sync_copy(h_hbm.at[pl.ds(base, ROWS)], rows_v)
        for i in range(ROWS // 16):
            s = seg_v[pl.ds(16 * i, 16)]
            p = pos_v[pl.ds(16 * i, 16)]
            idx_v[pl.ds(16 * i, 16)] = s * NPOS + p
        pltpu.sync_copy(rows_v, acc_sh.at[idx_v], add=True)
        pltpu.sync_copy(ones_v, cnt_sh.at[seg_v], add=True)
        return carry

    lax.fori_loop(0, nb_w, _batch, 0)

    plsc.subcore_barrier()

    pltpu.sync_copy(acc_sh.at[pl.ds(sid * STRIPE, STRIPE)],
                    out_hbm.at[cid, pl.ds(sid * STRIPE, STRIPE)])
    pltpu.sync_copy(cnt_sh.at[pl.ds(sid * 16, 16)],
                    cnt_out_hbm.at[cid, pl.ds(sid * 16, 16)])


_sc_call = functools.partial(
    pl.kernel,
    out_type=(
        jax.ShapeDtypeStruct((NC, ACC_ROWS, D), jnp.float32),
        jax.ShapeDtypeStruct((NC, B, D), jnp.float32),
    ),
    mesh=plsc.VectorSubcoreMesh(
        core_axis_name="c", subcore_axis_name="s",
        num_cores=NC, num_subcores=NS),
    scratch_types=[
        pltpu.VMEM((ROWS,), jnp.int32),
        pltpu.VMEM((ROWS,), jnp.int32),
        pltpu.VMEM((ROWS,), jnp.int32),
        pltpu.VMEM((ROWS, D), jnp.float32),
        pltpu.VMEM((ROWS, D), jnp.float32),
        pltpu.VMEM((STRIPE, D), jnp.float32),
        pltpu.VMEM_SHARED((ACC_ROWS, D), jnp.float32),
        pltpu.VMEM_SHARED((B, D), jnp.float32),
    ],
)(_sc_body)


def kernel(h, pos, segment_ids):
    partial, cnt = _sc_call(h, pos, segment_ids)
    acc = partial[0] + partial[1]
    counts = cnt[0, :, 0] + cnt[1, :, 0]
    norm = jnp.maximum(counts, 1.0)
    out = acc.reshape(B, NPOS, D) / norm[:, None, None]
    return out.reshape(B, NPOS * D)


# speedup vs baseline: 10.9322x; 1.5666x over previous
import functools

import jax
import jax.numpy as jnp
from jax import lax
from jax.experimental import pallas as pl
from jax.experimental.pallas import tpu as pltpu
from jax.experimental.pallas import tpu_sc as plsc

N = 100000
D = 128
B = 256
NPOS = 3
ROWS = 80
NC, NS = 2, 16
NW = NC * NS
NBW = 39
SPAN = NBW * ROWS
NX = (N - NW * SPAN) // ROWS
ACC_ROWS = B * NPOS
STRIPE = ACC_ROWS // NS


def _sc_body(h_hbm, pos_hbm, seg_hbm, out_hbm, cnt_out_hbm,
             seg_sp, pos_sp, idx_v, segb_v, seg_x, pos_x,
             rows_a, rows_b, ones_v, zbuf_v,
             sem_a, sem_b,
             acc_sh, cnt_sh):
    cid = lax.axis_index("c")
    sid = lax.axis_index("s")
    wid = sid * NC + cid
    span_base = wid * SPAN

    zeros16 = jnp.zeros((16,), jnp.float32)
    ones16 = jnp.ones((16,), jnp.float32)

    def _zrow(r, carry):
        for q in range(D // 16):
            zbuf_v[r, pl.ds(16 * q, 16)] = zeros16
        return carry
    lax.fori_loop(0, STRIPE, _zrow, 0)

    def _orow(r, carry):
        for q in range(D // 16):
            ones_v[r, pl.ds(16 * q, 16)] = ones16
        return carry
    lax.fori_loop(0, ROWS, _orow, 0)

    pltpu.sync_copy(zbuf_v, acc_sh.at[pl.ds(sid * STRIPE, STRIPE)])
    pltpu.sync_copy(zbuf_v.at[pl.ds(0, 16)], cnt_sh.at[pl.ds(sid * 16, 16)])

    pltpu.sync_copy(seg_hbm.at[pl.ds(span_base, SPAN)], seg_sp)
    pltpu.sync_copy(pos_hbm.at[pl.ds(span_base, SPAN)], pos_sp)

    plsc.subcore_barrier()

    def _wait(rows_r, sem_r):
        pltpu.make_async_copy(h_hbm.at[pl.ds(0, ROWS)], rows_r, sem_r).wait()

    def _scatter(j, rows_r):
        for i in range(ROWS // 16):
            off = j * ROWS + 16 * i
            s = seg_sp[pl.ds(off, 16)]
            p = pos_sp[pl.ds(off, 16)]
            idx_v[pl.ds(16 * i, 16)] = s * NPOS + p
            segb_v[pl.ds(16 * i, 16)] = s
        pltpu.sync_copy(rows_r, acc_sh.at[idx_v], add=True)
        pltpu.sync_copy(ones_v, cnt_sh.at[segb_v], add=True)

    pltpu.async_copy(h_hbm.at[pl.ds(span_base, ROWS)], rows_a, sem_a)

    def _pair(jj, carry):
        for par, rows_r, sem_r, rows_n, sem_n in (
                (0, rows_a, sem_a, rows_b, sem_b),
                (1, rows_b, sem_b, rows_a, sem_a)):
            j = 2 * jj + par
            pltpu.async_copy(
                h_hbm.at[pl.ds(span_base + (j + 1) * ROWS, ROWS)],
                rows_n, sem_n)
            _wait(rows_r, sem_r)
            _scatter(j, rows_r)
        return carry

    lax.fori_loop(0, (NBW - 1) // 2, _pair, 0)

    _wait(rows_a, sem_a)
    _scatter(NBW - 1, rows_a)

    @pl.when(wid < NX)
    def _():
        basex = NW * SPAN + wid * ROWS
        pltpu.sync_copy(seg_hbm.at[pl.ds(basex, ROWS)], seg_x)
        pltpu.sync_copy(pos_hbm.at[pl.ds(basex, ROWS)], pos_x)
        pltpu.sync_copy(h_hbm.at[pl.ds(basex, ROWS)], rows_a)
        for i in range(ROWS // 16):
            s = seg_x[pl.ds(16 * i, 16)]
            p = pos_x[pl.ds(16 * i, 16)]
            idx_v[pl.ds(16 * i, 16)] = s * NPOS + p
            segb_v[pl.ds(16 * i, 16)] = s
        pltpu.sync_copy(rows_a, acc_sh.at[idx_v], add=True)
        pltpu.sync_copy(ones_v, cnt_sh.at[segb_v], add=True)

    plsc.subcore_barrier()

    pltpu.sync_copy(acc_sh.at[pl.ds(sid * STRIPE, STRIPE)],
                    out_hbm.at[cid, pl.ds(sid * STRIPE, STRIPE)])
    pltpu.sync_copy(cnt_sh.at[pl.ds(sid * 16, 16)],
                    cnt_out_hbm.at[cid, pl.ds(sid * 16, 16)])


_sc_call = functools.partial(
    pl.kernel,
    out_type=(
        jax.ShapeDtypeStruct((NC, ACC_ROWS, D), jnp.float32),
        jax.ShapeDtypeStruct((NC, B, D), jnp.float32),
    ),
    mesh=plsc.VectorSubcoreMesh(
        core_axis_name="c", subcore_axis_name="s",
        num_cores=NC, num_subcores=NS),
    scratch_types=[
        pltpu.VMEM((SPAN,), jnp.int32),
        pltpu.VMEM((SPAN,), jnp.int32),
        pltpu.VMEM((ROWS,), jnp.int32),
        pltpu.VMEM((ROWS,), jnp.int32),
        pltpu.VMEM((ROWS,), jnp.int32),
        pltpu.VMEM((ROWS,), jnp.int32),
        pltpu.VMEM((ROWS, D), jnp.float32),
        pltpu.VMEM((ROWS, D), jnp.float32),
        pltpu.VMEM((ROWS, D), jnp.float32),
        pltpu.VMEM((STRIPE, D), jnp.float32),
        pltpu.SemaphoreType.DMA,
        pltpu.SemaphoreType.DMA,
        pltpu.VMEM_SHARED((ACC_ROWS, D), jnp.float32),
        pltpu.VMEM_SHARED((B, D), jnp.float32),
    ],
)(_sc_body)


def kernel(h, pos, segment_ids):
    partial, cnt = _sc_call(h, pos, segment_ids)
    acc = partial[0] + partial[1]
    counts = cnt[0, :, 0] + cnt[1, :, 0]
    norm = jnp.maximum(counts, 1.0)
    out = acc.reshape(B, NPOS, D) / norm[:, None, None]
    return out.reshape(B, NPOS * D)


# speedup vs baseline: 10.9747x; 1.0039x over previous
import functools

import jax
import jax.numpy as jnp
from jax import lax
from jax.experimental import pallas as pl
from jax.experimental.pallas import tpu as pltpu
from jax.experimental.pallas import tpu_sc as plsc

N = 100000
D = 128
B = 256
NPOS = 3
ROWS = 80
NC, NS = 2, 16
NW = NC * NS
NBW = 39
SPAN = NBW * ROWS
NX = (N - NW * SPAN) // ROWS
ACC_ROWS = B * NPOS
STRIPE = ACC_ROWS // NS


def _sc_body(h_hbm, pos_hbm, seg_hbm, out_hbm, cnt_out_hbm,
             seg_sp, pos_sp, idx_v, segb_v, seg_x, pos_x,
             rows_a, rows_b, ones_v, zbuf_v,
             sem_a, sem_b, sem_s,
             acc_sh, cnt_sh):
    cid = lax.axis_index("c")
    sid = lax.axis_index("s")
    wid = sid * NC + cid
    span_base = wid * SPAN

    zeros16 = jnp.zeros((16,), jnp.float32)
    ones16 = jnp.ones((16,), jnp.float32)

    def _zrow(r, carry):
        for q in range(D // 16):
            zbuf_v[r, pl.ds(16 * q, 16)] = zeros16
        return carry
    lax.fori_loop(0, STRIPE, _zrow, 0)

    def _orow(r, carry):
        for q in range(D // 16):
            ones_v[r, pl.ds(16 * q, 16)] = ones16
        return carry
    lax.fori_loop(0, ROWS, _orow, 0)

    pltpu.sync_copy(zbuf_v, acc_sh.at[pl.ds(sid * STRIPE, STRIPE)])
    pltpu.sync_copy(zbuf_v.at[pl.ds(0, 16)], cnt_sh.at[pl.ds(sid * 16, 16)])

    pltpu.sync_copy(seg_hbm.at[pl.ds(span_base, SPAN)], seg_sp)
    pltpu.sync_copy(pos_hbm.at[pl.ds(span_base, SPAN)], pos_sp)

    plsc.subcore_barrier()

    def _wait(rows_r, sem_r):
        pltpu.make_async_copy(h_hbm.at[pl.ds(0, ROWS)], rows_r, sem_r).wait()

    def _scatter(j, rows_r):
        for i in range(ROWS // 16):
            off = j * ROWS + 16 * i
            s = seg_sp[pl.ds(off, 16)]
            p = pos_sp[pl.ds(off, 16)]
            idx_v[pl.ds(16 * i, 16)] = s * NPOS + p
            segb_v[pl.ds(16 * i, 16)] = s
        d1 = pltpu.async_copy(rows_r, acc_sh.at[idx_v], sem_s, add=True)
        d2 = pltpu.async_copy(ones_v, cnt_sh.at[segb_v], sem_s, add=True)
        d1.wait()
        d2.wait()

    pltpu.async_copy(h_hbm.at[pl.ds(span_base, ROWS)], rows_a, sem_a)

    def _pair(jj, carry):
        for par, rows_r, sem_r, rows_n, sem_n in (
                (0, rows_a, sem_a, rows_b, sem_b),
                (1, rows_b, sem_b, rows_a, sem_a)):
            j = 2 * jj + par
            pltpu.async_copy(
                h_hbm.at[pl.ds(span_base + (j + 1) * ROWS, ROWS)],
                rows_n, sem_n)
            _wait(rows_r, sem_r)
            _scatter(j, rows_r)
        return carry

    lax.fori_loop(0, (NBW - 1) // 2, _pair, 0)

    _wait(rows_a, sem_a)
    _scatter(NBW - 1, rows_a)

    @pl.when(wid < NX)
    def _():
        basex = NW * SPAN + wid * ROWS
        pltpu.sync_copy(seg_hbm.at[pl.ds(basex, ROWS)], seg_x)
        pltpu.sync_copy(pos_hbm.at[pl.ds(basex, ROWS)], pos_x)
        pltpu.sync_copy(h_hbm.at[pl.ds(basex, ROWS)], rows_a)
        for i in range(ROWS // 16):
            s = seg_x[pl.ds(16 * i, 16)]
            p = pos_x[pl.ds(16 * i, 16)]
            idx_v[pl.ds(16 * i, 16)] = s * NPOS + p
            segb_v[pl.ds(16 * i, 16)] = s
        pltpu.sync_copy(rows_a, acc_sh.at[idx_v], add=True)
        pltpu.sync_copy(ones_v, cnt_sh.at[segb_v], add=True)

    plsc.subcore_barrier()

    pltpu.sync_copy(acc_sh.at[pl.ds(sid * STRIPE, STRIPE)],
                    out_hbm.at[cid, pl.ds(sid * STRIPE, STRIPE)])
    pltpu.sync_copy(cnt_sh.at[pl.ds(sid * 16, 16)],
                    cnt_out_hbm.at[cid, pl.ds(sid * 16, 16)])


_sc_call = functools.partial(
    pl.kernel,
    out_type=(
        jax.ShapeDtypeStruct((NC, ACC_ROWS, D), jnp.float32),
        jax.ShapeDtypeStruct((NC, B, D), jnp.float32),
    ),
    mesh=plsc.VectorSubcoreMesh(
        core_axis_name="c", subcore_axis_name="s",
        num_cores=NC, num_subcores=NS),
    scratch_types=[
        pltpu.VMEM((SPAN,), jnp.int32),
        pltpu.VMEM((SPAN,), jnp.int32),
        pltpu.VMEM((ROWS,), jnp.int32),
        pltpu.VMEM((ROWS,), jnp.int32),
        pltpu.VMEM((ROWS,), jnp.int32),
        pltpu.VMEM((ROWS,), jnp.int32),
        pltpu.VMEM((ROWS, D), jnp.float32),
        pltpu.VMEM((ROWS, D), jnp.float32),
        pltpu.VMEM((ROWS, D), jnp.float32),
        pltpu.VMEM((STRIPE, D), jnp.float32),
        pltpu.SemaphoreType.DMA,
        pltpu.SemaphoreType.DMA,
        pltpu.SemaphoreType.DMA,
        pltpu.VMEM_SHARED((ACC_ROWS, D), jnp.float32),
        pltpu.VMEM_SHARED((B, D), jnp.float32),
    ],
)(_sc_body)


def kernel(h, pos, segment_ids):
    partial, cnt = _sc_call(h, pos, segment_ids)
    acc = partial[0] + partial[1]
    counts = cnt[0, :, 0] + cnt[1, :, 0]
    norm = jnp.maximum(counts, 1.0)
    out = acc.reshape(B, NPOS, D) / norm[:, None, None]
    return out.reshape(B, NPOS * D)


# speedup vs baseline: 13.8977x; 1.2663x over previous
import functools

import jax
import jax.numpy as jnp
from jax import lax
from jax.experimental import pallas as pl
from jax.experimental.pallas import tpu as pltpu
from jax.experimental.pallas import tpu_sc as plsc

N = 100000
D = 128
B = 256
NPOS = 3
ROWS = 80
NC, NS = 2, 16
NW = NC * NS
NBW = 39
SPAN = NBW * ROWS
NX = (N - NW * SPAN) // ROWS
ACC_ROWS = B * NPOS
STRIPE = ACC_ROWS // NS
CROWS = 32


def _sc_body(h_hbm, pos_hbm, seg_hbm, out_hbm, cnt_out_hbm,
             seg_sp, pos_sp, idx_v, seg_x, pos_x,
             rows_a, rows_b, cl_v, zbuf_v,
             sem_a, sem_b, sem_s,
             acc_sh):
    cid = lax.axis_index("c")
    sid = lax.axis_index("s")
    wid = sid * NC + cid
    span_base = wid * SPAN

    zeros16 = jnp.zeros((16,), jnp.float32)

    def _zrow(r, carry):
        for q in range(D // 16):
            zbuf_v[r, pl.ds(16 * q, 16)] = zeros16
        return carry
    lax.fori_loop(0, STRIPE, _zrow, 0)
    for r in range(CROWS):
        for q in range(D // 16):
            cl_v[r, pl.ds(16 * q, 16)] = zeros16

    pltpu.sync_copy(zbuf_v, acc_sh.at[pl.ds(sid * STRIPE, STRIPE)])

    pltpu.sync_copy(seg_hbm.at[pl.ds(span_base, SPAN)], seg_sp)
    pltpu.sync_copy(pos_hbm.at[pl.ds(span_base, SPAN)], pos_sp)

    plsc.subcore_barrier()

    def _wait(rows_r, sem_r):
        pltpu.make_async_copy(h_hbm.at[pl.ds(0, ROWS)], rows_r, sem_r).wait()

    def _scatter(j, rows_r):
        for i in range(ROWS // 16):
            off = j * ROWS + 16 * i
            s = seg_sp[pl.ds(off, 16)]
            p = pos_sp[pl.ds(off, 16)]
            idx_v[pl.ds(16 * i, 16)] = s * NPOS + p
        pltpu.async_copy(rows_r, acc_sh.at[idx_v], sem_s, add=True).wait()

    pltpu.async_copy(h_hbm.at[pl.ds(span_base, ROWS)], rows_a, sem_a)

    def _pair(jj, carry):
        for par, rows_r, sem_r, rows_n, sem_n in (
                (0, rows_a, sem_a, rows_b, sem_b),
                (1, rows_b, sem_b, rows_a, sem_a)):
            j = 2 * jj + par
            pltpu.async_copy(
                h_hbm.at[pl.ds(span_base + (j + 1) * ROWS, ROWS)],
                rows_n, sem_n)
            _wait(rows_r, sem_r)
            _scatter(j, rows_r)
        return carry

    lax.fori_loop(0, (NBW - 1) // 2, _pair, 0)

    _wait(rows_a, sem_a)
    _scatter(NBW - 1, rows_a)

    bump1 = jnp.ones((16,), jnp.float32)
    bump16 = jnp.full((16,), 16.0, jnp.float32)

    def _bump(s, vec):
        r = lax.bitwise_and(s, 31)
        cb = lax.shift_right_logical(lax.bitwise_and(s, 224), 1)
        cl_v[r, pl.ds(cb, 16)] = cl_v[r, pl.ds(cb, 16)] + vec

    def _count16(ref, off):
        s_vec = ref[pl.ds(off, 16)]
        a = s_vec[0]
        b = s_vec[15]

        @pl.when(a == b)
        def _():
            _bump(a, bump16)

        @pl.when(a != b)
        def _():
            for l in range(16):
                _bump(s_vec[l], bump1)

    def _cchunk(ci, carry):
        _count16(seg_sp, ci * 16)
        return carry
    lax.fori_loop(0, SPAN // 16, _cchunk, 0)

    @pl.when(wid < NX)
    def _():
        basex = NW * SPAN + wid * ROWS
        pltpu.sync_copy(seg_hbm.at[pl.ds(basex, ROWS)], seg_x)
        pltpu.sync_copy(pos_hbm.at[pl.ds(basex, ROWS)], pos_x)
        pltpu.sync_copy(h_hbm.at[pl.ds(basex, ROWS)], rows_a)
        for i in range(ROWS // 16):
            s = seg_x[pl.ds(16 * i, 16)]
            p = pos_x[pl.ds(16 * i, 16)]
            idx_v[pl.ds(16 * i, 16)] = s * NPOS + p
        pltpu.sync_copy(rows_a, acc_sh.at[idx_v], add=True)
        for i in range(ROWS // 16):
            _count16(seg_x, i * 16)

    pltpu.sync_copy(cl_v, cnt_out_hbm.at[wid])

    plsc.subcore_barrier()

    pltpu.sync_copy(acc_sh.at[pl.ds(sid * STRIPE, STRIPE)],
                    out_hbm.at[cid, pl.ds(sid * STRIPE, STRIPE)])


_sc_call = functools.partial(
    pl.kernel,
    out_type=(
        jax.ShapeDtypeStruct((NC, ACC_ROWS, D), jnp.float32),
        jax.ShapeDtypeStruct((NW, CROWS, D), jnp.float32),
    ),
    mesh=plsc.VectorSubcoreMesh(
        core_axis_name="c", subcore_axis_name="s",
        num_cores=NC, num_subcores=NS),
    scratch_types=[
        pltpu.VMEM((SPAN,), jnp.int32),
        pltpu.VMEM((SPAN,), jnp.int32),
        pltpu.VMEM((ROWS,), jnp.int32),
        pltpu.VMEM((ROWS,), jnp.int32),
        pltpu.VMEM((ROWS,), jnp.int32),
        pltpu.VMEM((ROWS, D), jnp.float32),
        pltpu.VMEM((ROWS, D), jnp.float32),
        pltpu.VMEM((CROWS, D), jnp.float32),
        pltpu.VMEM((STRIPE, D), jnp.float32),
        pltpu.SemaphoreType.DMA,
        pltpu.SemaphoreType.DMA,
        pltpu.SemaphoreType.DMA,
        pltpu.VMEM_SHARED((ACC_ROWS, D), jnp.float32),
    ],
)(_sc_body)


def kernel(h, pos, segment_ids):
    partial, cnt = _sc_call(h, pos, segment_ids)
    acc = partial[0] + partial[1]
    counts = cnt.sum(axis=0).reshape(CROWS, 8, 16)[:, :, 0].T.reshape(B)
    norm = jnp.maximum(counts, 1.0)
    out = acc.reshape(B, NPOS, D) / norm[:, None, None]
    return out.reshape(B, NPOS * D)


# speedup vs baseline: 14.0991x; 1.0145x over previous
import functools

import jax
import jax.numpy as jnp
from jax import lax
from jax.experimental import pallas as pl
from jax.experimental.pallas import tpu as pltpu
from jax.experimental.pallas import tpu_sc as plsc

N = 100000
D = 128
B = 256
NPOS = 3
ROWS = 80
NC, NS = 2, 16
NW = NC * NS
NBW = 39
SPAN = NBW * ROWS
NX = (N - NW * SPAN) // ROWS
ACC_ROWS = B * NPOS
STRIPE = ACC_ROWS // NS
CROWS = 32


def _sc_body(h_hbm, pos_hbm, seg_hbm, out_hbm, cnt_out_hbm,
             seg_sp, pos_sp, seg_x, pos_x,
             rows_a, rows_b, rows_c, idx_a, idx_b, idx_c,
             cl_v, zbuf_v,
             sem_la, sem_lb, sem_lc, sem_sa, sem_sb, sem_sc3,
             acc_sh):
    cid = lax.axis_index("c")
    sid = lax.axis_index("s")
    wid = sid * NC + cid
    span_base = wid * SPAN

    zeros16 = jnp.zeros((16,), jnp.float32)
    bump1 = jnp.ones((16,), jnp.float32)
    bump16 = jnp.full((16,), 16.0, jnp.float32)

    def _zrow(r, carry):
        for q in range(D // 16):
            zbuf_v[r, pl.ds(16 * q, 16)] = zeros16
        return carry
    lax.fori_loop(0, STRIPE, _zrow, 0)

    def _crow(r, carry):
        for q in range(D // 16):
            cl_v[r, pl.ds(16 * q, 16)] = zeros16
        return carry
    lax.fori_loop(0, CROWS, _crow, 0)

    pltpu.sync_copy(zbuf_v, acc_sh.at[pl.ds(sid * STRIPE, STRIPE)])

    pltpu.sync_copy(seg_hbm.at[pl.ds(span_base, SPAN)], seg_sp)
    pltpu.sync_copy(pos_hbm.at[pl.ds(span_base, SPAN)], pos_sp)

    plsc.subcore_barrier()

    BUFS = ((rows_a, idx_a, sem_la, sem_sa),
            (rows_b, idx_b, sem_lb, sem_sb),
            (rows_c, idx_c, sem_lc, sem_sc3))

    def _load(j, b):
        rows_r, _, sem_l, _ = BUFS[b]
        pltpu.async_copy(h_hbm.at[pl.ds(span_base + j * ROWS, ROWS)],
                         rows_r, sem_l)

    def _wait_load(b):
        rows_r, _, sem_l, _ = BUFS[b]
        pltpu.make_async_copy(h_hbm.at[pl.ds(0, ROWS)], rows_r, sem_l).wait()

    def _drain(b):
        rows_r, idx_r, _, sem_s = BUFS[b]
        pltpu.make_async_copy(rows_r, acc_sh.at[idx_r], sem_s).wait()

    def _bump(s, vec):
        r = lax.bitwise_and(s, 31)
        cb = lax.shift_right_logical(lax.bitwise_and(s, 224), 1)
        cl_v[r, pl.ds(cb, 16)] = cl_v[r, pl.ds(cb, 16)] + vec

    def _count16(s_vec):
        a = s_vec[0]
        b = s_vec[15]

        @pl.when(a == b)
        def _():
            _bump(a, bump16)

        @pl.when(a != b)
        def _():
            for l in range(16):
                _bump(s_vec[l], bump1)

    def _process(j, b):
        rows_r, idx_r, _, sem_s = BUFS[b]
        for i in range(ROWS // 16):
            off = j * ROWS + 16 * i
            s = seg_sp[pl.ds(off, 16)]
            p = pos_sp[pl.ds(off, 16)]
            idx_r[pl.ds(16 * i, 16)] = s * NPOS + p
            _count16(s)
        pltpu.async_copy(rows_r, acc_sh.at[idx_r], sem_s, add=True)

    _load(0, 0)
    _load(1, 1)
    _wait_load(0)
    _process(0, 0)
    _load(2, 2)
    _wait_load(1)
    _process(1, 1)
    _drain(0)
    _load(3, 0)
    _wait_load(2)
    _process(2, 2)

    def _triple(jj, carry):
        for k in range(3):
            j0 = 3 * jj + k
            b = k
            _drain((k + 1) % 3)
            _load2 = span_base + (j0 + 1) * ROWS
            pltpu.async_copy(h_hbm.at[pl.ds(_load2, ROWS)],
                             BUFS[(k + 1) % 3][0], BUFS[(k + 1) % 3][2])
            _wait_load(b)
            _process(j0, b)
        return carry

    lax.fori_loop(1, 12, _triple, 0)

    _drain(1)
    _load(37, 1)
    _wait_load(0)
    _process(36, 0)
    _drain(2)
    _load(38, 2)
    _wait_load(1)
    _process(37, 1)
    _drain(0)
    _wait_load(2)
    _process(38, 2)
    _drain(1)
    _drain(2)

    @pl.when(wid < NX)
    def _():
        basex = NW * SPAN + wid * ROWS
        pltpu.sync_copy(seg_hbm.at[pl.ds(basex, ROWS)], seg_x)
        pltpu.sync_copy(pos_hbm.at[pl.ds(basex, ROWS)], pos_x)
        pltpu.sync_copy(h_hbm.at[pl.ds(basex, ROWS)], rows_a)
        for i in range(ROWS // 16):
            s = seg_x[pl.ds(16 * i, 16)]
            p = pos_x[pl.ds(16 * i, 16)]
            idx_a[pl.ds(16 * i, 16)] = s * NPOS + p
            _count16(s)
        pltpu.sync_copy(rows_a, acc_sh.at[idx_a], add=True)

    pltpu.sync_copy(cl_v, cnt_out_hbm.at[wid])

    plsc.subcore_barrier()

    pltpu.sync_copy(acc_sh.at[pl.ds(sid * STRIPE, STRIPE)],
                    out_hbm.at[cid, pl.ds(sid * STRIPE, STRIPE)])


_sc_call = functools.partial(
    pl.kernel,
    out_type=(
        jax.ShapeDtypeStruct((NC, ACC_ROWS, D), jnp.float32),
        jax.ShapeDtypeStruct((NW, CROWS, D), jnp.float32),
    ),
    mesh=plsc.VectorSubcoreMesh(
        core_axis_name="c", subcore_axis_name="s",
        num_cores=NC, num_subcores=NS),
    scratch_types=[
        pltpu.VMEM((SPAN,), jnp.int32),
        pltpu.VMEM((SPAN,), jnp.int32),
        pltpu.VMEM((ROWS,), jnp.int32),
        pltpu.VMEM((ROWS,), jnp.int32),
        pltpu.VMEM((ROWS, D), jnp.float32),
        pltpu.VMEM((ROWS, D), jnp.float32),
        pltpu.VMEM((ROWS, D), jnp.float32),
        pltpu.VMEM((ROWS,), jnp.int32),
        pltpu.VMEM((ROWS,), jnp.int32),
        pltpu.VMEM((ROWS,), jnp.int32),
        pltpu.VMEM((CROWS, D), jnp.float32),
        pltpu.VMEM((STRIPE, D), jnp.float32),
        pltpu.SemaphoreType.DMA,
        pltpu.SemaphoreType.DMA,
        pltpu.SemaphoreType.DMA,
        pltpu.SemaphoreType.DMA,
        pltpu.SemaphoreType.DMA,
        pltpu.SemaphoreType.DMA,
        pltpu.VMEM_SHARED((ACC_ROWS, D), jnp.float32),
    ],
)(_sc_body)


def kernel(h, pos, segment_ids):
    partial, cnt = _sc_call(h, pos, segment_ids)
    acc = partial[0] + partial[1]
    counts = cnt.sum(axis=0).reshape(CROWS, 8, 16)[:, :, 0].T.reshape(B)
    norm = jnp.maximum(counts, 1.0)
    out = acc.reshape(B, NPOS, D) / norm[:, None, None]
    return out.reshape(B, NPOS * D)
